# baseline (speedup 1.0000x reference)
import functools
import math

import jax
import jax.numpy as jnp
from jax import lax
from jax.experimental import pallas as pl
from jax.experimental.pallas import tpu as pltpu
from jax.experimental.pallas import tpu_sc as plsc

VOCAB = 1000000
D = 32
B = 4096
L = 50

NC = 2
NS = 16
NW = NC * NS
LANES = 16

CHUNK = B // NW
N_CHUNKS = L
NBUF = 10
N_GROUPS = N_CHUNKS // NBUF

SCALE = math.sqrt(D)


@functools.partial(
    pl.kernel,
    out_type=jax.ShapeDtypeStruct((L, B, D), jnp.float32),
    mesh=plsc.VectorSubcoreMesh(core_axis_name="c", subcore_axis_name="s"),
    scratch_types=[
        pltpu.VMEM((N_CHUNKS, CHUNK), jnp.int32),
        *[pltpu.VMEM((CHUNK, D), jnp.float32) for _ in range(NBUF)],
        *[pltpu.VMEM((CHUNK, D), jnp.float32) for _ in range(NBUF)],
        *[pltpu.SemaphoreType.DMA for _ in range(2 * NBUF)],
    ],
    compiler_params=pltpu.CompilerParams(use_tc_tiling_on_sc=False),
)
def _emb_lookup(xt_hbm, table_hbm, out_hbm, idx_v, *scratch):
    rows = scratch[:NBUF]
    stage = scratch[NBUF:2 * NBUF]
    gsem = scratch[2 * NBUF:3 * NBUF]
    ssem = scratch[3 * NBUF:]
    wid = lax.axis_index("s") * NC + lax.axis_index("c")
    pltpu.sync_copy(xt_hbm.at[:, pl.ds(wid * CHUNK, CHUNK)], idx_v)

    def out_at(c):
        return out_hbm.at[c, pl.ds(wid * CHUNK, CHUNK)]

    for b in range(NBUF):
        pltpu.async_copy(table_hbm.at[idx_v.at[b]], rows[b], gsem[b])

    @pl.loop(0, N_GROUPS)
    def _group(g):
        for b in range(NBUF):
            c = g * NBUF + b
            pltpu.make_async_copy(
                table_hbm.at[idx_v.at[c]], rows[b], gsem[b]).wait()

            @pl.when(g > 0)
            def _stage_free():
                pltpu.make_async_copy(stage[b], out_at(c), ssem[b]).wait()

            @pl.loop(0, CHUNK, unroll=8)
            def _row(r):
                for h in range(D // LANES):
                    sl = pl.ds(h * LANES, LANES)
                    stage[b][r, sl] = rows[b][r, sl] * SCALE

            @pl.when(g + 1 < N_GROUPS)
            def _prefetch():
                pltpu.async_copy(
                    table_hbm.at[idx_v.at[c + NBUF]], rows[b], gsem[b])

            pltpu.async_copy(stage[b], out_at(c), ssem[b])

    for b in range(NBUF):
        pltpu.make_async_copy(
            stage[b], out_hbm.at[0, pl.ds(wid * CHUNK, CHUNK)], ssem[b]).wait()


def kernel(x, table):
    out = _emb_lookup(x.T, table)
    return out.transpose(1, 0, 2)
